# baseline (speedup 1.0000x reference)
import jax
import jax.numpy as jnp
from jax import lax
from jax.experimental import pallas as pl
from jax.experimental.pallas import tpu as pltpu
from jax.experimental.pallas import tpu_sc as plsc

BATCH = 16384
DIM = 64
NC = 2
NS = 16
NW = NC * NS
BPW = BATCH // NW
CW = 128
CH = BPW // CW


def _gather_body(users_hbm, neg_hbm, ut_hbm, et_hbm,
                 u_out, n_out, uidx, nidx,
                 ubuf_a, ubuf_b, nbuf_a, nbuf_b,
                 sem_g_a, sem_g_b, sem_f_a, sem_f_b):
    wid = lax.axis_index("s") * NC + lax.axis_index("c")
    base = pl.multiple_of(wid * BPW, 128)
    for c in range(CH):
        pltpu.sync_copy(users_hbm.at[pl.ds(base + c * CW, CW)], uidx.at[c])
        pltpu.sync_copy(neg_hbm.at[pl.ds(base + c * CW, CW)], nidx.at[c])

    ubufs, nbufs = (ubuf_a, ubuf_b), (nbuf_a, nbuf_b)
    sem_g, sem_f = (sem_g_a, sem_g_b), (sem_f_a, sem_f_b)

    def gather(c, p):
        pltpu.async_copy(ut_hbm.at[uidx.at[c]], ubufs[p], sem_g[p])
        pltpu.async_copy(et_hbm.at[nidx.at[c]], nbufs[p], sem_g[p])

    def flush(c, p):
        pltpu.make_async_copy(ut_hbm.at[uidx.at[c]], ubufs[p],
                              sem_g[p]).wait()
        pltpu.make_async_copy(et_hbm.at[nidx.at[c]], nbufs[p],
                              sem_g[p]).wait()
        dst = pl.multiple_of(base + c * CW, 128)
        pltpu.async_copy(ubufs[p], u_out.at[pl.ds(dst, CW)], sem_f[p])
        pltpu.async_copy(nbufs[p], n_out.at[pl.ds(dst, CW)], sem_f[p])

    def wait_flush(c, p):
        dst = pl.multiple_of(base + c * CW, 128)
        pltpu.make_async_copy(ubufs[p], u_out.at[pl.ds(dst, CW)],
                              sem_f[p]).wait()
        pltpu.make_async_copy(nbufs[p], n_out.at[pl.ds(dst, CW)],
                              sem_f[p]).wait()

    gather(0, 0)
    for c in range(CH):
        p = c % 2
        if c + 1 < CH:
            gather(c + 1, 1 - p)
        if c >= 2:
            wait_flush(c - 2, p)
        flush(c, p)
    wait_flush(CH - 2, CH % 2)
    wait_flush(CH - 1, 1 - CH % 2)


_gather = pl.kernel(
    _gather_body,
    mesh=plsc.VectorSubcoreMesh(core_axis_name="c", subcore_axis_name="s"),
    out_type=(
        jax.ShapeDtypeStruct((BATCH, 2 * DIM), jnp.float32),
        jax.ShapeDtypeStruct((BATCH, 2 * DIM), jnp.float32),
    ),
    scratch_types=[
        pltpu.VMEM((CH, CW), jnp.int32),
        pltpu.VMEM((CH, CW), jnp.int32),
        pltpu.VMEM((CW, 2 * DIM), jnp.float32),
        pltpu.VMEM((CW, 2 * DIM), jnp.float32),
        pltpu.VMEM((CW, 2 * DIM), jnp.float32),
        pltpu.VMEM((CW, 2 * DIM), jnp.float32),
        pltpu.SemaphoreType.DMA,
        pltpu.SemaphoreType.DMA,
        pltpu.SemaphoreType.DMA,
        pltpu.SemaphoreType.DMA,
    ],
)


def _loss_body(up_ref, np_ref, u2_ref, n2_ref, out_ref):
    u = jnp.where(up_ref[...] == 1, u2_ref[:, DIM:], u2_ref[:, :DIM])
    n = jnp.where(np_ref[...] == 1, n2_ref[:, DIM:], n2_ref[:, :DIM])
    z = u * n
    sp = jnp.maximum(z, 0.0) + jnp.log1p(jnp.exp(-jnp.abs(z)))
    out_ref[0, 0] = jnp.mean(sp)


def kernel(users, pos, neg, user_table, entity_table):
    del pos
    users = users.astype(jnp.int32)
    neg = neg.astype(jnp.int32)
    ut2 = user_table.reshape(user_table.shape[0] // 2, 2 * DIM)
    et2 = entity_table.reshape(entity_table.shape[0] // 2, 2 * DIM)
    u2, n2 = _gather(users >> 1, neg >> 1, ut2, et2)
    loss = pl.pallas_call(
        _loss_body,
        out_shape=jax.ShapeDtypeStruct((1, 1), jnp.float32),
        out_specs=pl.BlockSpec(memory_space=pltpu.SMEM),
    )((users & 1).reshape(BATCH, 1), (neg & 1).reshape(BATCH, 1), u2, n2)
    return loss[0, 0]


# speedup vs baseline: 1.1219x; 1.1219x over previous
import jax
import jax.numpy as jnp
from jax import lax
from jax.experimental import pallas as pl
from jax.experimental.pallas import tpu as pltpu
from jax.experimental.pallas import tpu_sc as plsc

BATCH = 16384
DIM = 64
NC = 2
NS = 16
NW = NC * NS
BPW = BATCH // NW
CW = 128
CH = BPW // CW


def _gather_body(users_hbm, neg_hbm, ut_hbm, et_hbm,
                 u_out, n_out, uidx, nidx,
                 ubuf_a, ubuf_b, nbuf_a, nbuf_b,
                 sem_g_a, sem_g_b, sem_f_a, sem_f_b):
    wid = lax.axis_index("s") * NC + lax.axis_index("c")
    base = pl.multiple_of(wid * BPW, 128)
    for c in range(CH):
        pltpu.sync_copy(users_hbm.at[pl.ds(base + c * CW, CW)], uidx.at[c])
        pltpu.sync_copy(neg_hbm.at[pl.ds(base + c * CW, CW)], nidx.at[c])

    ubufs, nbufs = (ubuf_a, ubuf_b), (nbuf_a, nbuf_b)
    sem_g, sem_f = (sem_g_a, sem_g_b), (sem_f_a, sem_f_b)

    def gather(c, p):
        pltpu.async_copy(ut_hbm.at[uidx.at[c]], ubufs[p], sem_g[p])
        pltpu.async_copy(et_hbm.at[nidx.at[c]], nbufs[p], sem_g[p])

    def flush(c, p):
        pltpu.make_async_copy(ut_hbm.at[uidx.at[c]], ubufs[p],
                              sem_g[p]).wait()
        pltpu.make_async_copy(et_hbm.at[nidx.at[c]], nbufs[p],
                              sem_g[p]).wait()
        dst = pl.multiple_of(base + c * CW, 128)
        pltpu.async_copy(ubufs[p], u_out.at[pl.ds(dst, CW)], sem_f[p])
        pltpu.async_copy(nbufs[p], n_out.at[pl.ds(dst, CW)], sem_f[p])

    def wait_flush(c, p):
        dst = pl.multiple_of(base + c * CW, 128)
        pltpu.make_async_copy(ubufs[p], u_out.at[pl.ds(dst, CW)],
                              sem_f[p]).wait()
        pltpu.make_async_copy(nbufs[p], n_out.at[pl.ds(dst, CW)],
                              sem_f[p]).wait()

    gather(0, 0)
    for c in range(CH):
        p = c % 2
        if c + 1 < CH:
            gather(c + 1, 1 - p)
        if c >= 2:
            wait_flush(c - 2, p)
        flush(c, p)
    wait_flush(CH - 2, CH % 2)
    wait_flush(CH - 1, 1 - CH % 2)


_gather = pl.kernel(
    _gather_body,
    mesh=plsc.VectorSubcoreMesh(core_axis_name="c", subcore_axis_name="s"),
    out_type=(
        jax.ShapeDtypeStruct((BATCH, 2 * DIM), jnp.float32),
        jax.ShapeDtypeStruct((BATCH, 2 * DIM), jnp.float32),
    ),
    scratch_types=[
        pltpu.VMEM((CH, CW), jnp.int32),
        pltpu.VMEM((CH, CW), jnp.int32),
        pltpu.VMEM((CW, 2 * DIM), jnp.float32),
        pltpu.VMEM((CW, 2 * DIM), jnp.float32),
        pltpu.VMEM((CW, 2 * DIM), jnp.float32),
        pltpu.VMEM((CW, 2 * DIM), jnp.float32),
        pltpu.SemaphoreType.DMA,
        pltpu.SemaphoreType.DMA,
        pltpu.SemaphoreType.DMA,
        pltpu.SemaphoreType.DMA,
    ],
)


def _loss_body(u2_ref, n2_ref, out_ref):
    z = u2_ref[:, :DIM] * n2_ref[:, :DIM]
    sp = jnp.maximum(z, 0.0) + jnp.log1p(jnp.exp(-jnp.abs(z)))
    out_ref[0, 0] = jnp.mean(sp)


def kernel(users, pos, neg, user_table, entity_table):
    del pos
    users = users.astype(jnp.int32)
    neg = neg.astype(jnp.int32)
    ut2 = jnp.pad(user_table, ((0, 0), (0, DIM)))
    et2 = jnp.pad(entity_table, ((0, 0), (0, DIM)))
    u2, n2 = _gather(users, neg, ut2, et2)
    loss = pl.pallas_call(
        _loss_body,
        out_shape=jax.ShapeDtypeStruct((1, 1), jnp.float32),
        out_specs=pl.BlockSpec(memory_space=pltpu.SMEM),
    )(u2, n2)
    return loss[0, 0]
